# baseline (speedup 1.0000x reference)
import functools

import jax
import jax.numpy as jnp
from jax import lax
from jax.experimental import pallas as pl
from jax.experimental.pallas import tpu as pltpu
from jax.experimental.pallas import tpu_sc as plsc

N = 10000
E = 320000
D_IN = 128
H1 = 32
H2 = 64

NC = 2
NS = 16
NW = NC * NS

NP = 10240
EP = 327680
ROWS = EP // 128
RW = ROWS // NW
NPW = NP // NS

_MESH = plsc.VectorSubcoreMesh(core_axis_name="c", subcore_axis_name="s")
_SC_PARAMS = pltpu.CompilerParams(use_tc_tiling_on_sc=False)


@functools.partial(
    pl.kernel,
    out_type=jax.ShapeDtypeStruct((NC, NP), jnp.float32),
    mesh=_MESH,
    compiler_params=_SC_PARAMS,
    scratch_types=[
        pltpu.VMEM((RW, 128), jnp.int32),
        pltpu.VMEM((128,), jnp.float32),
        pltpu.VMEM((NPW,), jnp.float32),
        pltpu.VMEM_SHARED((NP,), jnp.float32),
    ],
)
def _sc_degree(dst_hbm, out_hbm, didx, ones, zeros, acc):
    c = lax.axis_index("c")
    s = lax.axis_index("s")
    w = s * NC + c

    for i in range(8):
        ones[pl.ds(i * 16, 16)] = jnp.ones((16,), jnp.float32)
    for i in range(NPW // 16):
        zeros[pl.ds(i * 16, 16)] = jnp.zeros((16,), jnp.float32)
    pltpu.sync_copy(zeros, acc.at[pl.ds(s * NPW, NPW)])
    plsc.subcore_barrier()

    pltpu.sync_copy(dst_hbm.at[pl.ds(w * RW, RW), :], didx)

    @pl.loop(0, RW)
    def _(j):
        pltpu.sync_copy(ones, acc.at[didx.at[j]], add=True)

    plsc.subcore_barrier()
    pltpu.sync_copy(acc.at[pl.ds(s * NPW, NPW)], out_hbm.at[c, pl.ds(s * NPW, NPW)])


CH = 512
EW = RW * 128
NCHUNK = EW // CH
SPC = CH // 128


def _make_sc_propagate(width, stage_y_in_spmem):
    scratch = [
        pltpu.VMEM((EW,), jnp.int32),
        pltpu.VMEM((RW, 128), jnp.int32),
        pltpu.VMEM((CH, width), jnp.float32),
        pltpu.VMEM((CH, width), jnp.float32),
        pltpu.VMEM_SHARED((NP, width), jnp.float32),
        pltpu.SemaphoreType.DMA,
        pltpu.SemaphoreType.DMA,
    ]
    if stage_y_in_spmem:
        scratch.insert(4, pltpu.VMEM_SHARED((NP, width), jnp.float32))

    @functools.partial(
        pl.kernel,
        out_type=jax.ShapeDtypeStruct((NC, NP, width), jnp.float32),
        mesh=_MESH,
        compiler_params=_SC_PARAMS,
        scratch_types=scratch,
    )
    def prop(y_hbm, src_hbm, dst_hbm, out_hbm, sidx, didx, rows0, rows1, *rest):
        if stage_y_in_spmem:
            y_sp, acc, sem0, sem1 = rest
        else:
            acc, sem0, sem1 = rest
            y_sp = None
        c = lax.axis_index("c")
        s = lax.axis_index("s")
        w = s * NC + c

        if stage_y_in_spmem:
            pltpu.sync_copy(y_hbm.at[pl.ds(s * NPW, NPW), :],
                            y_sp.at[pl.ds(s * NPW, NPW), :])
        ysrc = y_sp if stage_y_in_spmem else y_hbm
        pltpu.sync_copy(y_hbm.at[pl.ds(s * NPW, NPW), :], acc.at[pl.ds(s * NPW, NPW), :])
        plsc.subcore_barrier()

        pltpu.sync_copy(src_hbm.at[pl.ds(w * EW, EW)], sidx)
        pltpu.sync_copy(dst_hbm.at[pl.ds(w * RW, RW), :], didx)

        bufs = (rows0, rows1)
        sems = (sem0, sem1)
        pend = [None, None]
        pend[0] = pltpu.async_copy(ysrc.at[sidx.at[pl.ds(0, CH)]], rows0, sem0)
        for j in range(NCHUNK):
            if j + 1 < NCHUNK:
                pend[(j + 1) % 2] = pltpu.async_copy(
                    ysrc.at[sidx.at[pl.ds((j + 1) * CH, CH)]],
                    bufs[(j + 1) % 2], sems[(j + 1) % 2],
                )
            pend[j % 2].wait()
            for k in range(SPC):
                pltpu.sync_copy(
                    bufs[j % 2].at[pl.ds(k * 128, 128), :],
                    acc.at[didx.at[j * SPC + k]], add=True,
                )

        plsc.subcore_barrier()
        pltpu.sync_copy(
            acc.at[pl.ds(s * NPW, NPW), :], out_hbm.at[c, pl.ds(s * NPW, NPW), :]
        )

    return prop


_sc_prop32 = _make_sc_propagate(H1, stage_y_in_spmem=True)
_sc_prop64 = _make_sc_propagate(H2, stage_y_in_spmem=False)


_BR = 1024


def _tc1_body(x_ref, w1_ref, p0_ref, p1_ref, y_ref, dis_ref):
    deg = p0_ref[...] + p1_ref[...] + 1.0
    dis = lax.rsqrt(deg)
    xw = jnp.dot(x_ref[...], w1_ref[...], preferred_element_type=jnp.float32)
    y_ref[...] = xw * dis
    dis_ref[...] = dis


def _tc1(x_p, W1, p0, p1):
    return pl.pallas_call(
        _tc1_body,
        grid=(NP // _BR,),
        in_specs=[
            pl.BlockSpec((_BR, D_IN), lambda i: (i, 0)),
            pl.BlockSpec((D_IN, H1), lambda i: (0, 0)),
            pl.BlockSpec((_BR, 1), lambda i: (i, 0)),
            pl.BlockSpec((_BR, 1), lambda i: (i, 0)),
        ],
        out_specs=[
            pl.BlockSpec((_BR, H1), lambda i: (i, 0)),
            pl.BlockSpec((_BR, 1), lambda i: (i, 0)),
        ],
        out_shape=[
            jax.ShapeDtypeStruct((NP, H1), jnp.float32),
            jax.ShapeDtypeStruct((NP, 1), jnp.float32),
        ],
    )(x_p, W1, p0, p1)


def _tc2_body(a0_ref, a1_ref, y1_ref, dis_ref, w2_ref, b1_ref, y2_ref):
    dis = dis_ref[...]
    h1 = jnp.maximum(dis * (a0_ref[...] + a1_ref[...] - y1_ref[...]) + b1_ref[...], 0.0)
    y2_ref[...] = jnp.dot(h1, w2_ref[...], preferred_element_type=jnp.float32) * dis


def _tc2(a0, a1, y1, dis, W2, b1):
    return pl.pallas_call(
        _tc2_body,
        grid=(NP // _BR,),
        in_specs=[
            pl.BlockSpec((_BR, H1), lambda i: (i, 0)),
            pl.BlockSpec((_BR, H1), lambda i: (i, 0)),
            pl.BlockSpec((_BR, H1), lambda i: (i, 0)),
            pl.BlockSpec((_BR, 1), lambda i: (i, 0)),
            pl.BlockSpec((H1, H2), lambda i: (0, 0)),
            pl.BlockSpec((1, H1), lambda i: (0, 0)),
        ],
        out_specs=pl.BlockSpec((_BR, H2), lambda i: (i, 0)),
        out_shape=jax.ShapeDtypeStruct((NP, H2), jnp.float32),
    )(a0, a1, y1, dis, W2, b1)


_BR3 = 2000


def _tc3_body(a0_ref, a1_ref, y2_ref, dis_ref, b2_ref, wo_ref, bo_ref, o_ref):
    dis = dis_ref[...]
    h2 = jnp.maximum(dis * (a0_ref[...] + a1_ref[...] - y2_ref[...]) + b2_ref[...], 0.0)
    part = jnp.sum(h2 * wo_ref[...], keepdims=True)

    @pl.when(pl.program_id(0) == 0)
    def _():
        o_ref[...] = bo_ref[...]

    o_ref[...] += part


def _tc3(a0, a1, y2, dis, b2, Wo, bo):
    return pl.pallas_call(
        _tc3_body,
        grid=(N // _BR3,),
        in_specs=[
            pl.BlockSpec((_BR3, H2), lambda i: (i, 0)),
            pl.BlockSpec((_BR3, H2), lambda i: (i, 0)),
            pl.BlockSpec((_BR3, H2), lambda i: (i, 0)),
            pl.BlockSpec((_BR3, 1), lambda i: (i, 0)),
            pl.BlockSpec((1, H2), lambda i: (0, 0)),
            pl.BlockSpec((_BR3, H2), lambda i: (i, 0)),
            pl.BlockSpec((1, 1), lambda i: (0, 0)),
        ],
        out_specs=pl.BlockSpec((1, 1), lambda i: (0, 0)),
        out_shape=jax.ShapeDtypeStruct((1, 1), jnp.float32),
    )(a0, a1, y2, dis, b2, Wo, bo)


def kernel(x, edge_index, W1, b1, W2, b2, W_out, b_out):
    src = edge_index[0]
    dst = edge_index[1]
    pad = jnp.full((EP - E,), N, dtype=jnp.int32)
    src1 = jnp.concatenate([src, pad])
    dst2 = jnp.concatenate([dst, pad]).reshape(ROWS, 128)
    x_p = jnp.concatenate([x, jnp.zeros((NP - N, D_IN), jnp.float32)])

    degp = _sc_degree(dst2)
    p0 = degp[0].reshape(NP, 1)
    p1 = degp[1].reshape(NP, 1)

    y1, dis = _tc1(x_p, W1, p0, p1)
    acc1 = _sc_prop32(y1, src1, dst2)
    y2 = _tc2(acc1[0], acc1[1], y1, dis, W2, b1.reshape(1, H1))
    acc2 = _sc_prop64(y2, src1, dst2)
    out = _tc3(
        acc2[0], acc2[1], y2, dis,
        b2.reshape(1, H2), W_out.reshape(N, H2), b_out.reshape(1, 1),
    )
    return out


# speedup vs baseline: 1.3744x; 1.3744x over previous
import functools

import jax
import jax.numpy as jnp
from jax import lax
from jax.experimental import pallas as pl
from jax.experimental.pallas import tpu as pltpu
from jax.experimental.pallas import tpu_sc as plsc

N = 10000
E = 320000
D_IN = 128
H1 = 32
H2 = 64

NC = 2
NS = 16
NW = NC * NS

NP = 10240
EP = 327680
ROWS = EP // 128
RW = ROWS // NW
NPW = NP // NS

_MESH = plsc.VectorSubcoreMesh(core_axis_name="c", subcore_axis_name="s")
_SC_PARAMS = pltpu.CompilerParams(use_tc_tiling_on_sc=False)


@functools.partial(
    pl.kernel,
    out_type=jax.ShapeDtypeStruct((NC, NP), jnp.float32),
    mesh=_MESH,
    compiler_params=_SC_PARAMS,
    scratch_types=[
        pltpu.VMEM((RW, 128), jnp.int32),
        pltpu.VMEM((128,), jnp.float32),
        pltpu.VMEM((NPW,), jnp.float32),
        pltpu.VMEM_SHARED((NP,), jnp.float32),
    ],
)
def _sc_degree(dst_hbm, out_hbm, didx, ones, zeros, acc):
    c = lax.axis_index("c")
    s = lax.axis_index("s")
    w = s * NC + c

    for i in range(8):
        ones[pl.ds(i * 16, 16)] = jnp.ones((16,), jnp.float32)
    for i in range(NPW // 16):
        zeros[pl.ds(i * 16, 16)] = jnp.zeros((16,), jnp.float32)
    pltpu.sync_copy(zeros, acc.at[pl.ds(s * NPW, NPW)])
    plsc.subcore_barrier()

    pltpu.sync_copy(dst_hbm.at[pl.ds(w * RW, RW), :], didx)

    @pl.loop(0, RW)
    def _(j):
        pltpu.sync_copy(ones, acc.at[didx.at[j]], add=True)

    plsc.subcore_barrier()
    pltpu.sync_copy(acc.at[pl.ds(s * NPW, NPW)], out_hbm.at[c, pl.ds(s * NPW, NPW)])


CH = 512
EW = RW * 128
NCHUNK = EW // CH
SPC = CH // 128


def _edge_pipeline(y_sp, acc, sidx, didx, bufs, sems):
    pend = [None, None]
    pend[0] = pltpu.async_copy(y_sp.at[sidx.at[pl.ds(0, CH)]], bufs[0], sems[0])
    for j in range(NCHUNK):
        if j + 1 < NCHUNK:
            pend[(j + 1) % 2] = pltpu.async_copy(
                y_sp.at[sidx.at[pl.ds((j + 1) * CH, CH)]],
                bufs[(j + 1) % 2], sems[(j + 1) % 2],
            )
        pend[j % 2].wait()
        for k in range(SPC):
            pltpu.sync_copy(
                bufs[j % 2].at[pl.ds(k * 128, 128), :],
                acc.at[didx.at[j * SPC + k]], add=True,
            )


def _make_sc_propagate(nphase):
    @functools.partial(
        pl.kernel,
        out_type=[jax.ShapeDtypeStruct((NC, NP, H1), jnp.float32)] * nphase,
        mesh=_MESH,
        compiler_params=_SC_PARAMS,
        scratch_types=[
            pltpu.VMEM((EW,), jnp.int32),
            pltpu.VMEM((RW, 128), jnp.int32),
            pltpu.VMEM((CH, H1), jnp.float32),
            pltpu.VMEM((CH, H1), jnp.float32),
            pltpu.VMEM_SHARED((NP, H1), jnp.float32),
            pltpu.VMEM_SHARED((NP, H1), jnp.float32),
            pltpu.SemaphoreType.DMA,
            pltpu.SemaphoreType.DMA,
        ],
    )
    def prop(*refs):
        ys = refs[:nphase]
        src_hbm, dst_hbm = refs[nphase:nphase + 2]
        outs = refs[nphase + 2:2 * nphase + 2]
        sidx, didx, rows0, rows1, y_sp, acc, sem0, sem1 = refs[2 * nphase + 2:]
        c = lax.axis_index("c")
        s = lax.axis_index("s")
        w = s * NC + c
        nsl = pl.ds(s * NPW, NPW)

        pltpu.sync_copy(src_hbm.at[pl.ds(w * EW, EW)], sidx)
        pltpu.sync_copy(dst_hbm.at[pl.ds(w * RW, RW), :], didx)

        for p in range(nphase):
            pltpu.sync_copy(ys[p].at[nsl, :], y_sp.at[nsl, :])
            pltpu.sync_copy(ys[p].at[nsl, :], acc.at[nsl, :])
            plsc.subcore_barrier()
            _edge_pipeline(y_sp, acc, sidx, didx, (rows0, rows1), (sem0, sem1))
            plsc.subcore_barrier()
            pltpu.sync_copy(acc.at[nsl, :], outs[p].at[c, nsl, :])

    return prop


_sc_prop32 = _make_sc_propagate(1)
_sc_prop64 = _make_sc_propagate(2)


_BR = 1024


def _tc1_body(x_ref, w1_ref, p0_ref, p1_ref, y_ref, dis_ref):
    deg = p0_ref[...] + p1_ref[...] + 1.0
    dis = lax.rsqrt(deg)
    xw = jnp.dot(x_ref[...], w1_ref[...], preferred_element_type=jnp.float32)
    y_ref[...] = xw * dis
    dis_ref[...] = dis


def _tc1(x_p, W1, p0, p1):
    return pl.pallas_call(
        _tc1_body,
        grid=(NP // _BR,),
        in_specs=[
            pl.BlockSpec((_BR, D_IN), lambda i: (i, 0)),
            pl.BlockSpec((D_IN, H1), lambda i: (0, 0)),
            pl.BlockSpec((_BR, 1), lambda i: (i, 0)),
            pl.BlockSpec((_BR, 1), lambda i: (i, 0)),
        ],
        out_specs=[
            pl.BlockSpec((_BR, H1), lambda i: (i, 0)),
            pl.BlockSpec((_BR, 1), lambda i: (i, 0)),
        ],
        out_shape=[
            jax.ShapeDtypeStruct((NP, H1), jnp.float32),
            jax.ShapeDtypeStruct((NP, 1), jnp.float32),
        ],
    )(x_p, W1, p0, p1)


def _tc2_body(a0_ref, a1_ref, y1_ref, dis_ref, w2a_ref, w2b_ref, b1_ref,
              y2a_ref, y2b_ref):
    dis = dis_ref[...]
    h1 = jnp.maximum(dis * (a0_ref[...] + a1_ref[...] - y1_ref[...]) + b1_ref[...], 0.0)
    y2a_ref[...] = jnp.dot(h1, w2a_ref[...], preferred_element_type=jnp.float32) * dis
    y2b_ref[...] = jnp.dot(h1, w2b_ref[...], preferred_element_type=jnp.float32) * dis


def _tc2(a0, a1, y1, dis, W2a, W2b, b1):
    return pl.pallas_call(
        _tc2_body,
        grid=(NP // _BR,),
        in_specs=[
            pl.BlockSpec((_BR, H1), lambda i: (i, 0)),
            pl.BlockSpec((_BR, H1), lambda i: (i, 0)),
            pl.BlockSpec((_BR, H1), lambda i: (i, 0)),
            pl.BlockSpec((_BR, 1), lambda i: (i, 0)),
            pl.BlockSpec((H1, H1), lambda i: (0, 0)),
            pl.BlockSpec((H1, H1), lambda i: (0, 0)),
            pl.BlockSpec((1, H1), lambda i: (0, 0)),
        ],
        out_specs=[
            pl.BlockSpec((_BR, H1), lambda i: (i, 0)),
            pl.BlockSpec((_BR, H1), lambda i: (i, 0)),
        ],
        out_shape=[
            jax.ShapeDtypeStruct((NP, H1), jnp.float32),
            jax.ShapeDtypeStruct((NP, H1), jnp.float32),
        ],
    )(a0, a1, y1, dis, W2a, W2b, b1)


_BR3 = 2000


def _tc3_body(a0a_ref, a1a_ref, a0b_ref, a1b_ref, y2a_ref, y2b_ref, dis_ref,
              b2a_ref, b2b_ref, woa_ref, wob_ref, bo_ref, o_ref):
    dis = dis_ref[...]
    h2a = jnp.maximum(
        dis * (a0a_ref[...] + a1a_ref[...] - y2a_ref[...]) + b2a_ref[...], 0.0)
    h2b = jnp.maximum(
        dis * (a0b_ref[...] + a1b_ref[...] - y2b_ref[...]) + b2b_ref[...], 0.0)
    part = jnp.sum(h2a * woa_ref[...] + h2b * wob_ref[...], keepdims=True)

    @pl.when(pl.program_id(0) == 0)
    def _():
        o_ref[...] = bo_ref[...]

    o_ref[...] += part


def _tc3(a0a, a1a, a0b, a1b, y2a, y2b, dis, b2a, b2b, Woa, Wob, bo):
    row = pl.BlockSpec((_BR3, H1), lambda i: (i, 0))
    return pl.pallas_call(
        _tc3_body,
        grid=(N // _BR3,),
        in_specs=[
            row, row, row, row, row, row,
            pl.BlockSpec((_BR3, 1), lambda i: (i, 0)),
            pl.BlockSpec((1, H1), lambda i: (0, 0)),
            pl.BlockSpec((1, H1), lambda i: (0, 0)),
            row, row,
            pl.BlockSpec((1, 1), lambda i: (0, 0)),
        ],
        out_specs=pl.BlockSpec((1, 1), lambda i: (0, 0)),
        out_shape=jax.ShapeDtypeStruct((1, 1), jnp.float32),
    )(a0a, a1a, a0b, a1b, y2a, y2b, dis, b2a, b2b, Woa, Wob, bo)


def kernel(x, edge_index, W1, b1, W2, b2, W_out, b_out):
    src = edge_index[0]
    dst = edge_index[1]
    pad = jnp.full((EP - E,), N, dtype=jnp.int32)
    src1 = jnp.concatenate([src, pad])
    dst2 = jnp.concatenate([dst, pad]).reshape(ROWS, 128)
    x_p = jnp.concatenate([x, jnp.zeros((NP - N, D_IN), jnp.float32)])

    degp = _sc_degree(dst2)
    p0 = degp[0].reshape(NP, 1)
    p1 = degp[1].reshape(NP, 1)

    y1, dis = _tc1(x_p, W1, p0, p1)
    (acc1,) = _sc_prop32(y1, src1, dst2)
    y2a, y2b = _tc2(acc1[0], acc1[1], y1, dis, W2[:, :H1], W2[:, H1:],
                    b1.reshape(1, H1))
    acc2a, acc2b = _sc_prop64(y2a, y2b, src1, dst2)
    b2r = b2.reshape(1, H2)
    Wo = W_out.reshape(N, H2)
    out = _tc3(
        acc2a[0], acc2a[1], acc2b[0], acc2b[1], y2a, y2b, dis,
        b2r[:, :H1], b2r[:, H1:], Wo[:, :H1], Wo[:, H1:], b_out.reshape(1, 1),
    )
    return out


# speedup vs baseline: 1.3883x; 1.0101x over previous
import functools

import jax
import jax.numpy as jnp
from jax import lax
from jax.experimental import pallas as pl
from jax.experimental.pallas import tpu as pltpu
from jax.experimental.pallas import tpu_sc as plsc

N = 10000
E = 320000
D_IN = 128
H1 = 32
H2 = 64

NC = 2
NS = 16
NW = NC * NS

NP = 10240
EP = 327680
ROWS = EP // 128
RW = ROWS // NW
NPW = NP // NS

_MESH = plsc.VectorSubcoreMesh(core_axis_name="c", subcore_axis_name="s")
_SC_PARAMS = pltpu.CompilerParams(use_tc_tiling_on_sc=False)


@functools.partial(
    pl.kernel,
    out_type=jax.ShapeDtypeStruct((NC, NP), jnp.float32),
    mesh=_MESH,
    compiler_params=_SC_PARAMS,
    scratch_types=[
        pltpu.VMEM((RW, 128), jnp.int32),
        pltpu.VMEM((128,), jnp.float32),
        pltpu.VMEM((NPW,), jnp.float32),
        pltpu.VMEM_SHARED((NP,), jnp.float32),
    ],
)
def _sc_degree(dst_hbm, out_hbm, didx, ones, zeros, acc):
    c = lax.axis_index("c")
    s = lax.axis_index("s")
    w = s * NC + c

    for i in range(8):
        ones[pl.ds(i * 16, 16)] = jnp.ones((16,), jnp.float32)
    for i in range(NPW // 16):
        zeros[pl.ds(i * 16, 16)] = jnp.zeros((16,), jnp.float32)
    pltpu.sync_copy(zeros, acc.at[pl.ds(s * NPW, NPW)])
    plsc.subcore_barrier()

    pltpu.sync_copy(dst_hbm.at[pl.ds(w * RW, RW), :], didx)

    @pl.loop(0, RW)
    def _(j):
        pltpu.sync_copy(ones, acc.at[didx.at[j]], add=True)

    plsc.subcore_barrier()
    pltpu.sync_copy(acc.at[pl.ds(s * NPW, NPW)], out_hbm.at[c, pl.ds(s * NPW, NPW)])


CH = 1024
EW = RW * 128
NCHUNK = EW // CH
SPC = CH // 128


def _edge_pipeline(y_sp, acc, sidx, didx, bufs, sems):
    pend = [None, None]
    pend[0] = pltpu.async_copy(y_sp.at[sidx.at[pl.ds(0, CH)]], bufs[0], sems[0])
    for j in range(NCHUNK):
        if j + 1 < NCHUNK:
            pend[(j + 1) % 2] = pltpu.async_copy(
                y_sp.at[sidx.at[pl.ds((j + 1) * CH, CH)]],
                bufs[(j + 1) % 2], sems[(j + 1) % 2],
            )
        pend[j % 2].wait()
        for k in range(SPC):
            pltpu.sync_copy(
                bufs[j % 2].at[pl.ds(k * 128, 128), :],
                acc.at[didx.at[j * SPC + k]], add=True,
            )


def _make_sc_propagate(nphase):
    @functools.partial(
        pl.kernel,
        out_type=[jax.ShapeDtypeStruct((NC, NP, H1), jnp.float32)] * nphase,
        mesh=_MESH,
        compiler_params=_SC_PARAMS,
        scratch_types=[
            pltpu.VMEM((EW,), jnp.int32),
            pltpu.VMEM((RW, 128), jnp.int32),
            pltpu.VMEM((CH, H1), jnp.float32),
            pltpu.VMEM((CH, H1), jnp.float32),
            pltpu.VMEM_SHARED((NP, H1), jnp.float32),
            pltpu.VMEM_SHARED((NP, H1), jnp.float32),
            pltpu.SemaphoreType.DMA,
            pltpu.SemaphoreType.DMA,
        ],
    )
    def prop(*refs):
        ys = refs[:nphase]
        src_hbm, dst_hbm = refs[nphase:nphase + 2]
        outs = refs[nphase + 2:2 * nphase + 2]
        sidx, didx, rows0, rows1, y_sp, acc, sem0, sem1 = refs[2 * nphase + 2:]
        c = lax.axis_index("c")
        s = lax.axis_index("s")
        w = s * NC + c
        nsl = pl.ds(s * NPW, NPW)

        pltpu.sync_copy(src_hbm.at[pl.ds(w * EW, EW)], sidx)
        pltpu.sync_copy(dst_hbm.at[pl.ds(w * RW, RW), :], didx)

        for p in range(nphase):
            pltpu.sync_copy(ys[p].at[nsl, :], y_sp.at[nsl, :])
            pltpu.sync_copy(ys[p].at[nsl, :], acc.at[nsl, :])
            plsc.subcore_barrier()
            _edge_pipeline(y_sp, acc, sidx, didx, (rows0, rows1), (sem0, sem1))
            plsc.subcore_barrier()
            pltpu.sync_copy(acc.at[nsl, :], outs[p].at[c, nsl, :])

    return prop


_sc_prop32 = _make_sc_propagate(1)
_sc_prop64 = _make_sc_propagate(2)


_BR = 1024


def _tc1a_body(x_ref, w1_ref, xw_ref):
    xw_ref[...] = jnp.dot(x_ref[...], w1_ref[...], preferred_element_type=jnp.float32)


def _tc1a(x_p, W1):
    return pl.pallas_call(
        _tc1a_body,
        grid=(NP // _BR,),
        in_specs=[
            pl.BlockSpec((_BR, D_IN), lambda i: (i, 0)),
            pl.BlockSpec((D_IN, H1), lambda i: (0, 0)),
        ],
        out_specs=pl.BlockSpec((_BR, H1), lambda i: (i, 0)),
        out_shape=jax.ShapeDtypeStruct((NP, H1), jnp.float32),
    )(x_p, W1)


def _tc1b_body(xw_ref, p0_ref, p1_ref, y_ref, dis_ref):
    deg = p0_ref[...] + p1_ref[...] + 1.0
    dis = lax.rsqrt(deg)
    y_ref[...] = xw_ref[...] * dis
    dis_ref[...] = dis


def _tc1b(xw, p0, p1):
    return pl.pallas_call(
        _tc1b_body,
        grid=(NP // _BR,),
        in_specs=[
            pl.BlockSpec((_BR, H1), lambda i: (i, 0)),
            pl.BlockSpec((_BR, 1), lambda i: (i, 0)),
            pl.BlockSpec((_BR, 1), lambda i: (i, 0)),
        ],
        out_specs=[
            pl.BlockSpec((_BR, H1), lambda i: (i, 0)),
            pl.BlockSpec((_BR, 1), lambda i: (i, 0)),
        ],
        out_shape=[
            jax.ShapeDtypeStruct((NP, H1), jnp.float32),
            jax.ShapeDtypeStruct((NP, 1), jnp.float32),
        ],
    )(xw, p0, p1)


def _tc2_body(a0_ref, a1_ref, y1_ref, dis_ref, w2a_ref, w2b_ref, b1_ref,
              y2a_ref, y2b_ref):
    dis = dis_ref[...]
    h1 = jnp.maximum(dis * (a0_ref[...] + a1_ref[...] - y1_ref[...]) + b1_ref[...], 0.0)
    y2a_ref[...] = jnp.dot(h1, w2a_ref[...], preferred_element_type=jnp.float32) * dis
    y2b_ref[...] = jnp.dot(h1, w2b_ref[...], preferred_element_type=jnp.float32) * dis


def _tc2(a0, a1, y1, dis, W2a, W2b, b1):
    return pl.pallas_call(
        _tc2_body,
        grid=(NP // _BR,),
        in_specs=[
            pl.BlockSpec((_BR, H1), lambda i: (i, 0)),
            pl.BlockSpec((_BR, H1), lambda i: (i, 0)),
            pl.BlockSpec((_BR, H1), lambda i: (i, 0)),
            pl.BlockSpec((_BR, 1), lambda i: (i, 0)),
            pl.BlockSpec((H1, H1), lambda i: (0, 0)),
            pl.BlockSpec((H1, H1), lambda i: (0, 0)),
            pl.BlockSpec((1, H1), lambda i: (0, 0)),
        ],
        out_specs=[
            pl.BlockSpec((_BR, H1), lambda i: (i, 0)),
            pl.BlockSpec((_BR, H1), lambda i: (i, 0)),
        ],
        out_shape=[
            jax.ShapeDtypeStruct((NP, H1), jnp.float32),
            jax.ShapeDtypeStruct((NP, H1), jnp.float32),
        ],
    )(a0, a1, y1, dis, W2a, W2b, b1)


_BR3 = 2000


def _tc3_body(a0a_ref, a1a_ref, a0b_ref, a1b_ref, y2a_ref, y2b_ref, dis_ref,
              b2a_ref, b2b_ref, woa_ref, wob_ref, bo_ref, o_ref):
    dis = dis_ref[...]
    h2a = jnp.maximum(
        dis * (a0a_ref[...] + a1a_ref[...] - y2a_ref[...]) + b2a_ref[...], 0.0)
    h2b = jnp.maximum(
        dis * (a0b_ref[...] + a1b_ref[...] - y2b_ref[...]) + b2b_ref[...], 0.0)
    part = jnp.sum(h2a * woa_ref[...] + h2b * wob_ref[...], keepdims=True)

    @pl.when(pl.program_id(0) == 0)
    def _():
        o_ref[...] = bo_ref[...]

    o_ref[...] += part


def _tc3(a0a, a1a, a0b, a1b, y2a, y2b, dis, b2a, b2b, Woa, Wob, bo):
    row = pl.BlockSpec((_BR3, H1), lambda i: (i, 0))
    return pl.pallas_call(
        _tc3_body,
        grid=(N // _BR3,),
        in_specs=[
            row, row, row, row, row, row,
            pl.BlockSpec((_BR3, 1), lambda i: (i, 0)),
            pl.BlockSpec((1, H1), lambda i: (0, 0)),
            pl.BlockSpec((1, H1), lambda i: (0, 0)),
            row, row,
            pl.BlockSpec((1, 1), lambda i: (0, 0)),
        ],
        out_specs=pl.BlockSpec((1, 1), lambda i: (0, 0)),
        out_shape=jax.ShapeDtypeStruct((1, 1), jnp.float32),
    )(a0a, a1a, a0b, a1b, y2a, y2b, dis, b2a, b2b, Woa, Wob, bo)


def kernel(x, edge_index, W1, b1, W2, b2, W_out, b_out):
    src = edge_index[0]
    dst = edge_index[1]
    pad = jnp.full((EP - E,), N, dtype=jnp.int32)
    src1 = jnp.concatenate([src, pad])
    dst2 = jnp.concatenate([dst, pad]).reshape(ROWS, 128)
    x_p = jnp.concatenate([x, jnp.zeros((NP - N, D_IN), jnp.float32)])

    xw = _tc1a(x_p, W1)
    degp = _sc_degree(dst2)
    p0 = degp[0].reshape(NP, 1)
    p1 = degp[1].reshape(NP, 1)

    y1, dis = _tc1b(xw, p0, p1)
    (acc1,) = _sc_prop32(y1, src1, dst2)
    y2a, y2b = _tc2(acc1[0], acc1[1], y1, dis, W2[:, :H1], W2[:, H1:],
                    b1.reshape(1, H1))
    acc2a, acc2b = _sc_prop64(y2a, y2b, src1, dst2)
    b2r = b2.reshape(1, H2)
    Wo = W_out.reshape(N, H2)
    out = _tc3(
        acc2a[0], acc2a[1], acc2b[0], acc2b[1], y2a, y2b, dis,
        b2r[:, :H1], b2r[:, H1:], Wo[:, :H1], Wo[:, H1:], b_out.reshape(1, 1),
    )
    return out


# speedup vs baseline: 1.4063x; 1.0129x over previous
import functools

import jax
import jax.numpy as jnp
from jax import lax
from jax.experimental import pallas as pl
from jax.experimental.pallas import tpu as pltpu
from jax.experimental.pallas import tpu_sc as plsc

N = 10000
E = 320000
D_IN = 128
H1 = 32
H2 = 64

NC = 2
NS = 16
NW = NC * NS

NP = 10240
EP = 327680
ROWS = EP // 128
RW = ROWS // NW
NPW = NP // NS

_MESH = plsc.VectorSubcoreMesh(core_axis_name="c", subcore_axis_name="s")
_SC_PARAMS = pltpu.CompilerParams(use_tc_tiling_on_sc=False)


@functools.partial(
    pl.kernel,
    out_type=jax.ShapeDtypeStruct((NC, NP), jnp.float32),
    mesh=_MESH,
    compiler_params=_SC_PARAMS,
    scratch_types=[
        pltpu.VMEM((RW, 128), jnp.int32),
        pltpu.VMEM((128,), jnp.float32),
        pltpu.VMEM((NPW,), jnp.float32),
        pltpu.VMEM_SHARED((NP,), jnp.float32),
    ],
)
def _sc_degree(dst_hbm, out_hbm, didx, ones, zeros, acc):
    c = lax.axis_index("c")
    s = lax.axis_index("s")
    w = s * NC + c

    for i in range(8):
        ones[pl.ds(i * 16, 16)] = jnp.ones((16,), jnp.float32)
    for i in range(NPW // 16):
        zeros[pl.ds(i * 16, 16)] = jnp.zeros((16,), jnp.float32)
    pltpu.sync_copy(zeros, acc.at[pl.ds(s * NPW, NPW)])
    plsc.subcore_barrier()

    pltpu.sync_copy(dst_hbm.at[pl.ds(w * RW, RW), :], didx)

    @pl.loop(0, RW)
    def _(j):
        pltpu.sync_copy(ones, acc.at[didx.at[j]], add=True)

    plsc.subcore_barrier()
    pltpu.sync_copy(acc.at[pl.ds(s * NPW, NPW)], out_hbm.at[c, pl.ds(s * NPW, NPW)])


CH = 1024
ZR = 128
EW = RW * 128
NCHUNK = EW // CH
SPC = CH // 128


def _edge_pipeline(y_sp, acc, sidx, didx, bufs, sems):
    pend = [None, None]
    pend[0] = pltpu.async_copy(y_sp.at[sidx.at[pl.ds(0, CH)]], bufs[0], sems[0])
    for j in range(NCHUNK):
        if j + 1 < NCHUNK:
            pend[(j + 1) % 2] = pltpu.async_copy(
                y_sp.at[sidx.at[pl.ds((j + 1) * CH, CH)]],
                bufs[(j + 1) % 2], sems[(j + 1) % 2],
            )
        pend[j % 2].wait()
        for k in range(SPC):
            pltpu.sync_copy(
                bufs[j % 2].at[pl.ds(k * 128, 128), :],
                acc.at[didx.at[j * SPC + k]], add=True,
            )


def _make_sc_propagate(nphase):
    @functools.partial(
        pl.kernel,
        out_type=[jax.ShapeDtypeStruct((NC, NP, H1), jnp.float32)] * nphase,
        mesh=_MESH,
        compiler_params=_SC_PARAMS,
        scratch_types=[
            pltpu.VMEM((EW,), jnp.int32),
            pltpu.VMEM((RW, 128), jnp.int32),
            pltpu.VMEM((CH, H1), jnp.float32),
            pltpu.VMEM((CH, H1), jnp.float32),
            pltpu.VMEM((ZR, H1), jnp.float32),
            pltpu.VMEM_SHARED((NP, H1), jnp.float32),
            pltpu.VMEM_SHARED((NP, H1), jnp.float32),
            pltpu.SemaphoreType.DMA,
            pltpu.SemaphoreType.DMA,
        ],
    )
    def prop(*refs):
        ys = refs[:nphase]
        src_hbm, dst_hbm = refs[nphase:nphase + 2]
        outs = refs[nphase + 2:2 * nphase + 2]
        sidx, didx, rows0, rows1, zbuf, y_sp, acc, sem0, sem1 = refs[2 * nphase + 2:]
        c = lax.axis_index("c")
        s = lax.axis_index("s")
        w = s * NC + c
        nsl = pl.ds(s * NPW, NPW)

        pltpu.sync_copy(src_hbm.at[pl.ds(w * EW, EW)], sidx)
        pltpu.sync_copy(dst_hbm.at[pl.ds(w * RW, RW), :], didx)

        @pl.loop(0, ZR)
        def _(r):
            for q in range(H1 // 16):
                zbuf[r, pl.ds(q * 16, 16)] = jnp.zeros((16,), jnp.float32)

        for p in range(nphase):
            pltpu.sync_copy(ys[p].at[nsl, :], y_sp.at[nsl, :])
            for q in range(NPW // ZR):
                pltpu.sync_copy(
                    zbuf, acc.at[pl.ds(s * NPW + q * ZR, ZR), :])
            plsc.subcore_barrier()
            _edge_pipeline(y_sp, acc, sidx, didx, (rows0, rows1), (sem0, sem1))
            plsc.subcore_barrier()
            pltpu.sync_copy(acc.at[nsl, :], outs[p].at[c, nsl, :])

    return prop


_sc_prop32 = _make_sc_propagate(1)
_sc_prop64 = _make_sc_propagate(2)


_BR = 1024


def _tc1a_body(x_ref, w1_ref, xw_ref):
    xw_ref[...] = jnp.dot(x_ref[...], w1_ref[...], preferred_element_type=jnp.float32)


def _tc1a(x_p, W1):
    return pl.pallas_call(
        _tc1a_body,
        grid=(NP // _BR,),
        in_specs=[
            pl.BlockSpec((_BR, D_IN), lambda i: (i, 0)),
            pl.BlockSpec((D_IN, H1), lambda i: (0, 0)),
        ],
        out_specs=pl.BlockSpec((_BR, H1), lambda i: (i, 0)),
        out_shape=jax.ShapeDtypeStruct((NP, H1), jnp.float32),
    )(x_p, W1)


def _tc1b_body(xw_ref, p0_ref, p1_ref, y_ref, dis_ref):
    deg = p0_ref[...] + p1_ref[...] + 1.0
    dis = lax.rsqrt(deg)
    y_ref[...] = xw_ref[...] * dis
    dis_ref[...] = dis


def _tc1b(xw, p0, p1):
    return pl.pallas_call(
        _tc1b_body,
        grid=(NP // _BR,),
        in_specs=[
            pl.BlockSpec((_BR, H1), lambda i: (i, 0)),
            pl.BlockSpec((_BR, 1), lambda i: (i, 0)),
            pl.BlockSpec((_BR, 1), lambda i: (i, 0)),
        ],
        out_specs=[
            pl.BlockSpec((_BR, H1), lambda i: (i, 0)),
            pl.BlockSpec((_BR, 1), lambda i: (i, 0)),
        ],
        out_shape=[
            jax.ShapeDtypeStruct((NP, H1), jnp.float32),
            jax.ShapeDtypeStruct((NP, 1), jnp.float32),
        ],
    )(xw, p0, p1)


def _tc2_body(a0_ref, a1_ref, y1_ref, dis_ref, w2a_ref, w2b_ref, b1_ref,
              y2a_ref, y2b_ref):
    dis = dis_ref[...]
    h1 = jnp.maximum(dis * (a0_ref[...] + a1_ref[...] + y1_ref[...]) + b1_ref[...], 0.0)
    y2a_ref[...] = jnp.dot(h1, w2a_ref[...], preferred_element_type=jnp.float32) * dis
    y2b_ref[...] = jnp.dot(h1, w2b_ref[...], preferred_element_type=jnp.float32) * dis


def _tc2(a0, a1, y1, dis, W2a, W2b, b1):
    return pl.pallas_call(
        _tc2_body,
        grid=(NP // _BR,),
        in_specs=[
            pl.BlockSpec((_BR, H1), lambda i: (i, 0)),
            pl.BlockSpec((_BR, H1), lambda i: (i, 0)),
            pl.BlockSpec((_BR, H1), lambda i: (i, 0)),
            pl.BlockSpec((_BR, 1), lambda i: (i, 0)),
            pl.BlockSpec((H1, H1), lambda i: (0, 0)),
            pl.BlockSpec((H1, H1), lambda i: (0, 0)),
            pl.BlockSpec((1, H1), lambda i: (0, 0)),
        ],
        out_specs=[
            pl.BlockSpec((_BR, H1), lambda i: (i, 0)),
            pl.BlockSpec((_BR, H1), lambda i: (i, 0)),
        ],
        out_shape=[
            jax.ShapeDtypeStruct((NP, H1), jnp.float32),
            jax.ShapeDtypeStruct((NP, H1), jnp.float32),
        ],
    )(a0, a1, y1, dis, W2a, W2b, b1)


_BR3 = 2000


def _tc3_body(a0a_ref, a1a_ref, a0b_ref, a1b_ref, y2a_ref, y2b_ref, dis_ref,
              b2a_ref, b2b_ref, woa_ref, wob_ref, bo_ref, o_ref):
    dis = dis_ref[...]
    h2a = jnp.maximum(
        dis * (a0a_ref[...] + a1a_ref[...] + y2a_ref[...]) + b2a_ref[...], 0.0)
    h2b = jnp.maximum(
        dis * (a0b_ref[...] + a1b_ref[...] + y2b_ref[...]) + b2b_ref[...], 0.0)
    part = jnp.sum(h2a * woa_ref[...] + h2b * wob_ref[...], keepdims=True)

    @pl.when(pl.program_id(0) == 0)
    def _():
        o_ref[...] = bo_ref[...]

    o_ref[...] += part


def _tc3(a0a, a1a, a0b, a1b, y2a, y2b, dis, b2a, b2b, Woa, Wob, bo):
    row = pl.BlockSpec((_BR3, H1), lambda i: (i, 0))
    return pl.pallas_call(
        _tc3_body,
        grid=(N // _BR3,),
        in_specs=[
            row, row, row, row, row, row,
            pl.BlockSpec((_BR3, 1), lambda i: (i, 0)),
            pl.BlockSpec((1, H1), lambda i: (0, 0)),
            pl.BlockSpec((1, H1), lambda i: (0, 0)),
            row, row,
            pl.BlockSpec((1, 1), lambda i: (0, 0)),
        ],
        out_specs=pl.BlockSpec((1, 1), lambda i: (0, 0)),
        out_shape=jax.ShapeDtypeStruct((1, 1), jnp.float32),
    )(a0a, a1a, a0b, a1b, y2a, y2b, dis, b2a, b2b, Woa, Wob, bo)


def kernel(x, edge_index, W1, b1, W2, b2, W_out, b_out):
    src = edge_index[0]
    dst = edge_index[1]
    pad = jnp.full((EP - E,), N, dtype=jnp.int32)
    src1 = jnp.concatenate([src, pad])
    dst2 = jnp.concatenate([dst, pad]).reshape(ROWS, 128)
    x_p = jnp.concatenate([x, jnp.zeros((NP - N, D_IN), jnp.float32)])

    xw = _tc1a(x_p, W1)
    degp = _sc_degree(dst2)
    p0 = degp[0].reshape(NP, 1)
    p1 = degp[1].reshape(NP, 1)

    y1, dis = _tc1b(xw, p0, p1)
    (acc1,) = _sc_prop32(y1, src1, dst2)
    y2a, y2b = _tc2(acc1[0], acc1[1], y1, dis, W2[:, :H1], W2[:, H1:],
                    b1.reshape(1, H1))
    acc2a, acc2b = _sc_prop64(y2a, y2b, src1, dst2)
    b2r = b2.reshape(1, H2)
    Wo = W_out.reshape(N, H2)
    out = _tc3(
        acc2a[0], acc2a[1], acc2b[0], acc2b[1], y2a, y2b, dis,
        b2r[:, :H1], b2r[:, H1:], Wo[:, :H1], Wo[:, H1:], b_out.reshape(1, 1),
    )
    return out
